# baseline (speedup 1.0000x reference)
import functools

import jax
import jax.numpy as jnp
from jax import lax
from jax.experimental import pallas as pl
from jax.experimental.pallas import tpu as pltpu
from jax.experimental.pallas import tpu_sc as plsc

D_MODEL = 64
D_VOCAB = 1000000
D_PAD = 128


_TBLK = 512


_SUB = 128


def _transpose_body(w_ref, out_ref):
    w = w_ref[...]
    r = lax.broadcasted_iota(jnp.int32, (_SUB, _SUB), 0)
    c = lax.broadcasted_iota(jnp.int32, (_SUB, _SUB), 1)
    eye = (r == c).astype(jnp.float32)
    zeros = jnp.zeros((_TBLK, D_PAD - D_MODEL), jnp.float32)
    parts = []
    for j in range(_TBLK // _SUB):
        sub = w[:, j * _SUB:(j + 1) * _SUB]
        parts.append(lax.dot_general(
            eye, sub, (((1,), (1,)), ((), ())),
            precision=lax.Precision.HIGHEST,
            preferred_element_type=jnp.float32))
    out_ref[...] = jnp.concatenate(
        [jnp.concatenate(parts, axis=0), zeros], axis=1)


def _transpose_table(W_E):
    return pl.pallas_call(
        _transpose_body,
        grid=(pl.cdiv(D_VOCAB, _TBLK),),
        in_specs=[pl.BlockSpec((D_MODEL, _TBLK), lambda i: (0, i))],
        out_specs=pl.BlockSpec((_TBLK, D_PAD), lambda i: (i, 0)),
        out_shape=jax.ShapeDtypeStruct((D_VOCAB, D_PAD), jnp.float32),
    )(W_E)


_CHUNK = 512


def _make_gather(B):
    info = plsc.get_sparse_core_info()
    NW = info.num_cores * info.num_subcores
    b_per_w = B // NW
    n_chunks = b_per_w // _CHUNK
    mesh = plsc.VectorSubcoreMesh(core_axis_name="c", subcore_axis_name="s")

    @functools.partial(
        pl.kernel,
        mesh=mesh,
        compiler_params=pltpu.CompilerParams(use_tc_tiling_on_sc=True),
        out_type=jax.ShapeDtypeStruct((B, D_PAD), jnp.float32),
        scratch_types=[
            pltpu.VMEM((_CHUNK,), jnp.int32),
            pltpu.VMEM((_CHUNK, D_PAD), jnp.float32),
            pltpu.SemaphoreType.DMA,
        ],
    )
    def gather_kernel(table_hbm, idx_hbm, out_hbm, idx_v, rows_v, sem):
        wid = lax.axis_index("s") * info.num_cores + lax.axis_index("c")
        wbase = wid * b_per_w

        def body(c, carry):
            base = wbase + c * _CHUNK
            pltpu.sync_copy(idx_hbm.at[pl.ds(base, _CHUNK)], idx_v)
            pltpu.async_copy(table_hbm.at[idx_v], rows_v, sem).wait()
            pltpu.sync_copy(rows_v, out_hbm.at[pl.ds(base, _CHUNK)])
            return carry

        lax.fori_loop(0, n_chunks, body, 0)

    return gather_kernel


def kernel(x, W_E):
    b, p = x.shape
    W_T = _transpose_table(W_E)
    idx = x.reshape(-1).astype(jnp.int32)
    out = _make_gather(b * p)(W_T, idx)
    return out[:, :D_MODEL].reshape(b, p, D_MODEL)


# speedup vs baseline: 1.8937x; 1.8937x over previous
import functools

import jax
import jax.numpy as jnp
from jax import lax
from jax.experimental import pallas as pl
from jax.experimental.pallas import tpu as pltpu
from jax.experimental.pallas import tpu_sc as plsc

D_MODEL = 64
D_VOCAB = 1000000
D_PAD = 128


_TBLK = 4096


_SUB = 128


def _transpose_body(w_ref, out_ref):
    w = w_ref[...]
    r = lax.broadcasted_iota(jnp.int32, (_SUB, _SUB), 0)
    c = lax.broadcasted_iota(jnp.int32, (_SUB, _SUB), 1)
    eye = (r == c).astype(jnp.float32)
    zeros = jnp.zeros((_TBLK, D_PAD - D_MODEL), jnp.float32)
    parts = []
    for j in range(_TBLK // _SUB):
        sub = w[:, j * _SUB:(j + 1) * _SUB]
        parts.append(lax.dot_general(
            eye, sub, (((1,), (1,)), ((), ())),
            precision=lax.Precision.HIGHEST,
            preferred_element_type=jnp.float32))
    out_ref[...] = jnp.concatenate(
        [jnp.concatenate(parts, axis=0), zeros], axis=1)


def _transpose_table(W_E):
    return pl.pallas_call(
        _transpose_body,
        grid=(pl.cdiv(D_VOCAB, _TBLK),),
        in_specs=[pl.BlockSpec((D_MODEL, _TBLK), lambda i: (0, i))],
        out_specs=pl.BlockSpec((_TBLK, D_PAD), lambda i: (i, 0)),
        out_shape=jax.ShapeDtypeStruct((D_VOCAB, D_PAD), jnp.float32),
    )(W_E)


_CHUNK = 512


def _make_gather(B):
    info = plsc.get_sparse_core_info()
    NW = info.num_cores * info.num_subcores
    b_per_w = B // NW
    n_chunks = b_per_w // _CHUNK
    mesh = plsc.VectorSubcoreMesh(core_axis_name="c", subcore_axis_name="s")

    @functools.partial(
        pl.kernel,
        mesh=mesh,
        compiler_params=pltpu.CompilerParams(use_tc_tiling_on_sc=True),
        out_type=jax.ShapeDtypeStruct((B, D_PAD), jnp.float32),
        scratch_types=[
            pltpu.VMEM((_CHUNK,), jnp.int32),
            pltpu.VMEM((_CHUNK, D_PAD), jnp.float32),
            pltpu.SemaphoreType.DMA,
        ],
    )
    def gather_kernel(table_hbm, idx_hbm, out_hbm, idx_v, rows_v, sem):
        wid = lax.axis_index("s") * info.num_cores + lax.axis_index("c")
        wbase = wid * b_per_w

        def body(c, carry):
            base = wbase + c * _CHUNK
            pltpu.sync_copy(idx_hbm.at[pl.ds(base, _CHUNK)], idx_v)
            pltpu.async_copy(table_hbm.at[idx_v], rows_v, sem).wait()
            pltpu.sync_copy(rows_v, out_hbm.at[pl.ds(base, _CHUNK)])
            return carry

        lax.fori_loop(0, n_chunks, body, 0)

    return gather_kernel


def kernel(x, W_E):
    b, p = x.shape
    W_T = _transpose_table(W_E)
    idx = x.reshape(-1).astype(jnp.int32)
    out = _make_gather(b * p)(W_T, idx)
    return out[:, :D_MODEL].reshape(b, p, D_MODEL)


# speedup vs baseline: 2.0736x; 1.0950x over previous
import functools

import jax
import jax.numpy as jnp
from jax import lax
from jax.experimental import pallas as pl
from jax.experimental.pallas import tpu as pltpu
from jax.experimental.pallas import tpu_sc as plsc

D_MODEL = 64
D_VOCAB = 1000000
D_PAD = 128


_TBLK = 8192


_SUB = 128


def _transpose_body(w_ref, out_ref):
    w = w_ref[...]
    r = lax.broadcasted_iota(jnp.int32, (_SUB, _SUB), 0)
    c = lax.broadcasted_iota(jnp.int32, (_SUB, _SUB), 1)
    eye = (r == c).astype(jnp.float32)
    zeros = jnp.zeros((_TBLK, D_PAD - D_MODEL), jnp.float32)
    parts = []
    for j in range(_TBLK // _SUB):
        sub = w[:, j * _SUB:(j + 1) * _SUB]
        parts.append(lax.dot_general(
            eye, sub, (((1,), (1,)), ((), ())),
            precision=lax.Precision.HIGHEST,
            preferred_element_type=jnp.float32))
    out_ref[...] = jnp.concatenate(
        [jnp.concatenate(parts, axis=0), zeros], axis=1)


def _transpose_table(W_E):
    return pl.pallas_call(
        _transpose_body,
        grid=(pl.cdiv(D_VOCAB, _TBLK),),
        in_specs=[pl.BlockSpec((D_MODEL, _TBLK), lambda i: (0, i))],
        out_specs=pl.BlockSpec((_TBLK, D_PAD), lambda i: (i, 0)),
        out_shape=jax.ShapeDtypeStruct((D_VOCAB, D_PAD), jnp.float32),
    )(W_E)


_CHUNK = 320


def _make_gather(B):
    info = plsc.get_sparse_core_info()
    NW = info.num_cores * info.num_subcores
    b_per_w = B // NW
    n_pairs = b_per_w // (2 * _CHUNK)
    mesh = plsc.VectorSubcoreMesh(core_axis_name="c", subcore_axis_name="s")

    @functools.partial(
        pl.kernel,
        mesh=mesh,
        compiler_params=pltpu.CompilerParams(use_tc_tiling_on_sc=True),
        out_type=jax.ShapeDtypeStruct((B, D_PAD), jnp.float32),
        scratch_types=[
            pltpu.VMEM((b_per_w,), jnp.int32),
            pltpu.VMEM((_CHUNK, D_PAD), jnp.float32),
            pltpu.VMEM((_CHUNK, D_PAD), jnp.float32),
            pltpu.SemaphoreType.DMA,
            pltpu.SemaphoreType.DMA,
            pltpu.SemaphoreType.DMA,
            pltpu.SemaphoreType.DMA,
        ],
    )
    def gather_kernel(table_hbm, idx_hbm, out_hbm, idx_v, rows0, rows1,
                      gsem0, gsem1, wsem0, wsem1):
        wid = lax.axis_index("s") * info.num_cores + lax.axis_index("c")
        wbase = wid * b_per_w
        pltpu.sync_copy(idx_hbm.at[pl.ds(wbase, b_per_w)], idx_v)

        def step(c, rows, gsem, wsem, phase):
            off = (2 * c + phase) * _CHUNK

            @pl.when(c > 0)
            def _():
                pltpu.make_async_copy(
                    rows, out_hbm.at[pl.ds(0, _CHUNK)], wsem).wait()

            pltpu.async_copy(
                table_hbm.at[idx_v.at[pl.ds(off, _CHUNK)]], rows, gsem
            ).wait()
            pltpu.async_copy(rows, out_hbm.at[pl.ds(wbase + off, _CHUNK)],
                             wsem)

        def body(c, carry):
            step(c, rows0, gsem0, wsem0, 0)
            step(c, rows1, gsem1, wsem1, 1)
            return carry

        lax.fori_loop(0, n_pairs, body, 0)
        pltpu.make_async_copy(rows0, out_hbm.at[pl.ds(0, _CHUNK)], wsem0).wait()
        pltpu.make_async_copy(rows1, out_hbm.at[pl.ds(0, _CHUNK)], wsem1).wait()

    return gather_kernel


def kernel(x, W_E):
    b, p = x.shape
    W_T = _transpose_table(W_E)
    idx = x.reshape(-1).astype(jnp.int32)
    out = _make_gather(b * p)(W_T, idx)
    return out[:, :D_MODEL].reshape(b, p, D_MODEL)


# speedup vs baseline: 2.0938x; 1.0097x over previous
import functools

import jax
import jax.numpy as jnp
from jax import lax
from jax.experimental import pallas as pl
from jax.experimental.pallas import tpu as pltpu
from jax.experimental.pallas import tpu_sc as plsc

D_MODEL = 64
D_VOCAB = 1000000
D_PAD = 128


_TBLK = 16384


_SUB = 128


def _transpose_body(w_ref, out_ref):
    w = w_ref[...]
    r = lax.broadcasted_iota(jnp.int32, (_SUB, _SUB), 0)
    c = lax.broadcasted_iota(jnp.int32, (_SUB, _SUB), 1)
    eye = (r == c).astype(jnp.float32)
    zeros = jnp.zeros((_TBLK, D_PAD - D_MODEL), jnp.float32)
    parts = []
    for j in range(_TBLK // _SUB):
        sub = w[:, j * _SUB:(j + 1) * _SUB]
        parts.append(lax.dot_general(
            eye, sub, (((1,), (1,)), ((), ())),
            precision=lax.Precision.HIGHEST,
            preferred_element_type=jnp.float32))
    out_ref[...] = jnp.concatenate(
        [jnp.concatenate(parts, axis=0), zeros], axis=1)


def _transpose_table(W_E):
    return pl.pallas_call(
        _transpose_body,
        grid=(pl.cdiv(D_VOCAB, _TBLK),),
        in_specs=[pl.BlockSpec((D_MODEL, _TBLK), lambda i: (0, i))],
        out_specs=pl.BlockSpec((_TBLK, D_PAD), lambda i: (i, 0)),
        out_shape=jax.ShapeDtypeStruct((D_VOCAB, D_PAD), jnp.float32),
    )(W_E)


_CHUNK = 320


def _make_gather(B):
    info = plsc.get_sparse_core_info()
    NW = info.num_cores * info.num_subcores
    b_per_w = B // NW
    n_pairs = b_per_w // (2 * _CHUNK)
    mesh = plsc.VectorSubcoreMesh(core_axis_name="c", subcore_axis_name="s")

    @functools.partial(
        pl.kernel,
        mesh=mesh,
        compiler_params=pltpu.CompilerParams(use_tc_tiling_on_sc=True),
        out_type=jax.ShapeDtypeStruct((B, D_PAD), jnp.float32),
        scratch_types=[
            pltpu.VMEM((b_per_w,), jnp.int32),
            pltpu.VMEM((_CHUNK, D_PAD), jnp.float32),
            pltpu.VMEM((_CHUNK, D_PAD), jnp.float32),
            pltpu.SemaphoreType.DMA,
            pltpu.SemaphoreType.DMA,
            pltpu.SemaphoreType.DMA,
            pltpu.SemaphoreType.DMA,
        ],
    )
    def gather_kernel(table_hbm, idx_hbm, out_hbm, idx_v, rows0, rows1,
                      gsem0, gsem1, wsem0, wsem1):
        wid = lax.axis_index("s") * info.num_cores + lax.axis_index("c")
        wbase = wid * b_per_w
        pltpu.sync_copy(idx_hbm.at[pl.ds(wbase, b_per_w)], idx_v)

        def step(c, rows, gsem, wsem, phase):
            off = (2 * c + phase) * _CHUNK

            @pl.when(c > 0)
            def _():
                pltpu.make_async_copy(
                    rows, out_hbm.at[pl.ds(0, _CHUNK)], wsem).wait()

            pltpu.async_copy(
                table_hbm.at[idx_v.at[pl.ds(off, _CHUNK)]], rows, gsem
            ).wait()
            pltpu.async_copy(rows, out_hbm.at[pl.ds(wbase + off, _CHUNK)],
                             wsem)

        def body(c, carry):
            step(c, rows0, gsem0, wsem0, 0)
            step(c, rows1, gsem1, wsem1, 1)
            return carry

        lax.fori_loop(0, n_pairs, body, 0)
        pltpu.make_async_copy(rows0, out_hbm.at[pl.ds(0, _CHUNK)], wsem0).wait()
        pltpu.make_async_copy(rows1, out_hbm.at[pl.ds(0, _CHUNK)], wsem1).wait()

    return gather_kernel


def kernel(x, W_E):
    b, p = x.shape
    W_T = _transpose_table(W_E)
    idx = x.reshape(-1).astype(jnp.int32)
    out = _make_gather(b * p)(W_T, idx)
    return out[:, :D_MODEL].reshape(b, p, D_MODEL)
